# Initial kernel scaffold; baseline (speedup 1.0000x reference)
import jax
import jax.numpy as jnp
from jax.experimental import pallas as pl


def kernel(positions, domain):
    raise NotImplementedError("write your pallas kernel here")


# speedup vs baseline: 2.1778x; 2.1778x over previous
import functools

import jax
import jax.numpy as jnp
from jax.experimental import pallas as pl

_RADIUS = 0.1
_FACTORS = (0.25, 0.5, 0.75)
_TM = 256


def _adj_kernel(si_ref, sj_ref, out_ref, *, tm: int, n: int, r2: float):
    i = pl.program_id(0)
    best = None
    for s in range(4):
        d2 = None
        for c in range(3):
            idx = s * 3 + c
            diff = si_ref[:, idx:idx + 1] - sj_ref[idx:idx + 1, :]
            sq = diff * diff
            d2 = sq if d2 is None else d2 + sq
        best = d2 if best is None else jnp.minimum(best, d2)
    row_ids = jax.lax.broadcasted_iota(jnp.int32, (tm, n), 0) + i * tm
    col_ids = jax.lax.broadcasted_iota(jnp.int32, (tm, n), 1)
    adj = (best <= r2) & (row_ids != col_ids)
    out_ref[...] = adj.astype(jnp.float32)


def kernel(positions, domain):
    n = positions.shape[0]
    shifted = [positions]
    for f in _FACTORS:
        shifted.append(jnp.remainder(positions + f * domain, domain))
    si = jnp.concatenate(shifted, axis=1)
    si = jnp.pad(si, ((0, 0), (0, 4)))
    sj = si.T

    grid = (n // _TM,)
    body = functools.partial(_adj_kernel, tm=_TM, n=n, r2=_RADIUS * _RADIUS)
    out = pl.pallas_call(
        body,
        grid=grid,
        in_specs=[
            pl.BlockSpec((_TM, 16), lambda i: (i, 0)),
            pl.BlockSpec((16, n), lambda i: (0, 0)),
        ],
        out_specs=pl.BlockSpec((_TM, n), lambda i: (i, 0)),
        out_shape=jax.ShapeDtypeStruct((n, n), jnp.float32),
    )(si, sj)
    return out


# speedup vs baseline: 3.2920x; 1.5116x over previous
import functools

import jax
import jax.numpy as jnp
from jax.experimental import pallas as pl
from jax.experimental.pallas import tpu as pltpu

_RADIUS = 0.1
_FACTORS = (0.25, 0.5, 0.75)
_B = 512


def _adj_kernel(imap_ref, jmap_ref, si_ref, sj_ref, out_ref,
                scratch, scratch_t, sems, *, b: int, steps: int, r2: float):
    t = pl.program_id(0)
    i = imap_ref[t]
    j = jmap_ref[t]
    slot = jax.lax.rem(t, 2)

    def copies(s):
        row = pl.ds(imap_ref[t - 2 * s] * b, b)
        col = pl.ds(jmap_ref[t - 2 * s] * b, b)
        c0 = pltpu.make_async_copy(
            scratch.at[slot], out_ref.at[row, col], sems.at[slot, 0])
        c1 = pltpu.make_async_copy(
            scratch_t.at[slot], out_ref.at[col, row], sems.at[slot, 1])
        return c0, c1

    @pl.when(t >= 2)
    def _():
        c0, c1 = copies(1)
        c0.wait()
        c1.wait()

    best = None
    for s in range(4):
        d2 = None
        for c in range(3):
            idx = s * 3 + c
            diff = si_ref[:, idx:idx + 1] - sj_ref[idx:idx + 1, :]
            sq = diff * diff
            d2 = sq if d2 is None else d2 + sq
        best = d2 if best is None else jnp.minimum(best, d2)
    adj = (best <= r2).astype(jnp.float32)
    scratch[slot] = adj

    @pl.when(i == j)
    def _():
        rid = jax.lax.broadcasted_iota(jnp.int32, (b, b), 0)
        cid = jax.lax.broadcasted_iota(jnp.int32, (b, b), 1)
        scratch[slot] = jnp.where(rid == cid, 0.0, scratch[slot])

    scratch_t[slot] = scratch[slot].T

    c0, c1 = copies(0)
    c0.start()
    c1.start()

    @pl.when(t == steps - 1)
    def _():
        c0, c1 = copies(0)
        c0.wait()
        c1.wait()

    @pl.when(t == steps - 2)
    def _():
        c0, c1 = copies(0)
        c0.wait()
        c1.wait()


def kernel(positions, domain):
    n = positions.shape[0]
    shifted = [positions]
    for f in _FACTORS:
        shifted.append(jnp.remainder(positions + f * domain, domain))
    si = jnp.concatenate(shifted, axis=1)
    si = jnp.pad(si, ((0, 0), (0, 4)))
    sj = si.T

    nt = n // _B
    pairs = [(i, j) for i in range(nt) for j in range(i, nt)]
    steps = len(pairs)
    imap = jnp.asarray([p[0] for p in pairs], dtype=jnp.int32)
    jmap = jnp.asarray([p[1] for p in pairs], dtype=jnp.int32)

    body = functools.partial(_adj_kernel, b=_B, steps=steps,
                             r2=_RADIUS * _RADIUS)
    grid_spec = pltpu.PrefetchScalarGridSpec(
        num_scalar_prefetch=2,
        grid=(steps,),
        in_specs=[
            pl.BlockSpec((_B, 16), lambda t, im, jm: (im[t], 0)),
            pl.BlockSpec((16, _B), lambda t, im, jm: (0, jm[t])),
        ],
        out_specs=pl.BlockSpec(memory_space=pl.ANY),
        scratch_shapes=[
            pltpu.MemorySpace.VMEM((2, _B, _B), jnp.float32),
            pltpu.MemorySpace.VMEM((2, _B, _B), jnp.float32),
            pltpu.SemaphoreType.DMA((2, 2)),
        ],
    )
    out = pl.pallas_call(
        body,
        grid_spec=grid_spec,
        out_shape=jax.ShapeDtypeStruct((n, n), jnp.float32),
    )(imap, jmap, si, sj)
    return out


# speedup vs baseline: 3.4773x; 1.0563x over previous
import functools

import jax
import jax.numpy as jnp
from jax.experimental import pallas as pl
from jax.experimental.pallas import tpu as pltpu

_RADIUS = 0.1
_FACTORS = (0.25, 0.5, 0.75)
_B = 512


def _adj_kernel(imap_ref, jmap_ref, si_ref, sj_ref, out_ref,
                scratch, scratch_t, sems, *, b: int, steps: int, r2: float):
    t = pl.program_id(0)
    i = imap_ref[t]
    j = jmap_ref[t]
    slot = jax.lax.rem(t, 2)

    def copies(dt):
        tt = t - dt
        s = jax.lax.rem(tt, 2)
        row = pl.ds(imap_ref[tt] * b, b)
        col = pl.ds(jmap_ref[tt] * b, b)
        c0 = pltpu.make_async_copy(
            scratch.at[s], out_ref.at[row, col], sems.at[s, 0])
        c1 = pltpu.make_async_copy(
            scratch_t.at[s], out_ref.at[col, row], sems.at[s, 1])
        return c0, c1

    @pl.when(t >= 2)
    def _():
        c0, c1 = copies(2)
        c0.wait()
        c1.wait()

    best = None
    for s in range(4):
        d2 = None
        for c in range(3):
            idx = s * 3 + c
            diff = si_ref[:, idx:idx + 1] - sj_ref[idx:idx + 1, :]
            sq = diff * diff
            d2 = sq if d2 is None else d2 + sq
        best = d2 if best is None else jnp.minimum(best, d2)
    adj = (best <= r2).astype(jnp.float32)

    @pl.when(i == j)
    def _():
        rid = jax.lax.broadcasted_iota(jnp.int32, (b, b), 0)
        cid = jax.lax.broadcasted_iota(jnp.int32, (b, b), 1)
        masked = jnp.where(rid == cid, 0.0, adj)
        scratch[slot] = masked
        scratch_t[slot] = masked

    @pl.when(i != j)
    def _():
        scratch[slot] = adj
        scratch_t[slot] = adj.T

    c0, c1 = copies(0)
    c0.start()
    c1.start()

    @pl.when(t == steps - 1)
    def _():
        c0, c1 = copies(0)
        c0.wait()
        c1.wait()
        p0, p1 = copies(1)
        p0.wait()
        p1.wait()


def kernel(positions, domain):
    n = positions.shape[0]
    shifted = [positions]
    for f in _FACTORS:
        shifted.append(jnp.remainder(positions + f * domain, domain))
    si = jnp.concatenate(shifted, axis=1)
    si = jnp.pad(si, ((0, 0), (0, 4)))
    sj = si.T

    nt = n // _B
    pairs = [(i, j) for i in range(nt) for j in range(i, nt)]
    steps = len(pairs)
    imap = jnp.asarray([p[0] for p in pairs], dtype=jnp.int32)
    jmap = jnp.asarray([p[1] for p in pairs], dtype=jnp.int32)

    body = functools.partial(_adj_kernel, b=_B, steps=steps,
                             r2=_RADIUS * _RADIUS)
    grid_spec = pltpu.PrefetchScalarGridSpec(
        num_scalar_prefetch=2,
        grid=(steps,),
        in_specs=[
            pl.BlockSpec((_B, 16), lambda t, im, jm: (im[t], 0)),
            pl.BlockSpec((16, _B), lambda t, im, jm: (0, jm[t])),
        ],
        out_specs=pl.BlockSpec(memory_space=pl.ANY),
        scratch_shapes=[
            pltpu.MemorySpace.VMEM((2, _B, _B), jnp.float32),
            pltpu.MemorySpace.VMEM((2, _B, _B), jnp.float32),
            pltpu.SemaphoreType.DMA((2, 2)),
        ],
    )
    out = pl.pallas_call(
        body,
        grid_spec=grid_spec,
        out_shape=jax.ShapeDtypeStruct((n, n), jnp.float32),
    )(imap, jmap, si, sj)
    return out
